# baseline (device time: 52299 ns/iter reference)
import jax
import jax.numpy as jnp
from jax import lax
from jax.experimental import pallas as pl
from jax.experimental.pallas import tpu as pltpu

N_DEV = 4
B = 2
SQ = 256
SKV_SHARD = 256
HQ = 4
DH = 64
WINDOW = 128
NEG = -1e9


def kernel(x, Wq, K_ext, V_ext, Wo):
    K2 = K_ext.reshape(B, SKV_SHARD, HQ * DH)
    V2 = V_ext.reshape(B, SKV_SHARD, HQ * DH)

    def body(x_ref, wq_ref, k_ref, v_ref, wo_ref, out_ref,
             comm_o, comm_ml, send_o, recv_o, send_ml, recv_ml):
        my_pos = lax.axis_index("i")
        left = lax.rem(my_pos + (N_DEV - 1), N_DEV)
        right = lax.rem(my_pos + 1, N_DEV)

        barrier_sem = pltpu.get_barrier_semaphore()
        for nbr in (left, right):
            pl.semaphore_signal(
                barrier_sem, inc=1,
                device_id=(nbr,), device_id_type=pl.DeviceIdType.MESH,
            )
        pl.semaphore_wait(barrier_sem, 2)

        kv_off = my_pos * SKV_SHARD
        qi = lax.broadcasted_iota(jnp.int32, (SQ, SKV_SHARD), 0)
        kj = lax.broadcasted_iota(jnp.int32, (SQ, SKV_SHARD), 1) + kv_off
        mask = jnp.abs(qi - kj) <= WINDOW

        for b in range(B):
            xb = x_ref[b]
            qb = jnp.dot(xb, wq_ref[...], preferred_element_type=jnp.float32)
            kb = k_ref[b]
            vb = v_ref[b]
            for h in range(HQ):
                bh = b * HQ + h
                qbh = qb[:, h * DH:(h + 1) * DH]
                kbh = kb[:, h * DH:(h + 1) * DH]
                vbh = vb[:, h * DH:(h + 1) * DH]
                s = lax.dot_general(
                    qbh, kbh, (((1,), (1,)), ((), ())),
                    preferred_element_type=jnp.float32,
                ) * 0.125
                s = jnp.where(mask, s, NEG)
                m = jnp.max(s, axis=1)
                w = jnp.exp(s - m[:, None])
                l = jnp.sum(w, axis=1)
                o = jnp.dot(w, vbh, preferred_element_type=jnp.float32)
                comm_o[0, bh] = o
                comm_ml[0, bh] = m
                comm_ml[0, B * HQ + bh] = l

        for hop in range(N_DEV - 1):
            rd_o = pltpu.make_async_remote_copy(
                src_ref=comm_o.at[hop],
                dst_ref=comm_o.at[hop + 1],
                send_sem=send_o.at[hop],
                recv_sem=recv_o.at[hop],
                device_id=(right,),
                device_id_type=pl.DeviceIdType.MESH,
            )
            rd_ml = pltpu.make_async_remote_copy(
                src_ref=comm_ml.at[hop],
                dst_ref=comm_ml.at[hop + 1],
                send_sem=send_ml.at[hop],
                recv_sem=recv_ml.at[hop],
                device_id=(right,),
                device_id_type=pl.DeviceIdType.MESH,
            )
            rd_o.start()
            rd_ml.start()
            rd_o.wait()
            rd_ml.wait()

        m_run = comm_ml[0, 0:B * HQ, :]
        l_run = comm_ml[0, B * HQ:2 * B * HQ, :]
        o_run = comm_o[0]
        for c in range(1, N_DEV):
            m_c = comm_ml[c, 0:B * HQ, :]
            l_c = comm_ml[c, B * HQ:2 * B * HQ, :]
            o_c = comm_o[c]
            m_new = jnp.maximum(m_run, m_c)
            fa = jnp.exp(m_run - m_new)
            fb = jnp.exp(m_c - m_new)
            o_run = fa[:, :, None] * o_run + fb[:, :, None] * o_c
            l_run = fa * l_run + fb * l_c
            m_run = m_new
        ctx = o_run / l_run[:, :, None]

        for b in range(B):
            ctx_b = jnp.concatenate(
                [ctx[b * HQ + h] for h in range(HQ)], axis=1
            )
            out_ref[b] = jnp.dot(
                ctx_b, wo_ref[...], preferred_element_type=jnp.float32
            )

    return pl.pallas_call(
        body,
        out_shape=jax.ShapeDtypeStruct((B, SQ, 512), jnp.float32),
        in_specs=[pl.BlockSpec(memory_space=pltpu.VMEM)] * 5,
        out_specs=pl.BlockSpec(memory_space=pltpu.VMEM),
        scratch_shapes=[
            pltpu.VMEM((N_DEV, B * HQ, SQ, DH), jnp.float32),
            pltpu.VMEM((N_DEV, 2 * B * HQ, SQ), jnp.float32),
            pltpu.SemaphoreType.DMA((N_DEV - 1,)),
            pltpu.SemaphoreType.DMA((N_DEV - 1,)),
            pltpu.SemaphoreType.DMA((N_DEV - 1,)),
            pltpu.SemaphoreType.DMA((N_DEV - 1,)),
        ],
        compiler_params=pltpu.CompilerParams(collective_id=0),
    )(x, Wq, K2, V2, Wo)


# device time: 36164 ns/iter; 1.4462x vs baseline; 1.4462x over previous
import jax
import jax.numpy as jnp
from jax import lax
from jax.experimental import pallas as pl
from jax.experimental.pallas import tpu as pltpu

N_DEV = 4
B = 2
SQ = 256
SKV_SHARD = 256
HQ = 4
DH = 64
WINDOW = 128
NEG = -1e9
BH = B * HQ


def kernel(x, Wq, K_ext, V_ext, Wo):
    K2 = K_ext.reshape(B, SKV_SHARD, HQ * DH)
    V2 = V_ext.reshape(B, SKV_SHARD, HQ * DH)

    def body(x_ref, wq_ref, k_ref, v_ref, wo_ref, out_ref,
             o_buf, ml_buf, ctx_buf, ex_send, ex_recv, cx_send, cx_recv):
        my_pos = lax.axis_index("i")
        left = lax.rem(my_pos + (N_DEV - 1), N_DEV)
        right = lax.rem(my_pos + 1, N_DEV)
        peer = (N_DEV - 1) - my_pos
        is01 = my_pos <= 1

        barrier_sem = pltpu.get_barrier_semaphore()
        for nbr in (left, right):
            pl.semaphore_signal(
                barrier_sem, inc=1,
                device_id=(nbr,), device_id_type=pl.DeviceIdType.MESH,
            )
        pl.semaphore_wait(barrier_sem, 2)

        @pl.when(is01)
        def _producer():
            kv_off = my_pos * SKV_SHARD
            qi = lax.broadcasted_iota(jnp.int32, (SQ, SKV_SHARD), 0)
            kj = lax.broadcasted_iota(jnp.int32, (SQ, SKV_SHARD), 1) + kv_off
            mask = jnp.abs(qi - kj) <= WINDOW

            for b in range(B):
                xb = x_ref[b]
                qb = jnp.dot(xb, wq_ref[...],
                             preferred_element_type=jnp.float32)
                kb = k_ref[b]
                vb = v_ref[b]
                for h in range(HQ):
                    bh = b * HQ + h
                    qbh = qb[:, h * DH:(h + 1) * DH]
                    kbh = kb[:, h * DH:(h + 1) * DH]
                    vbh = vb[:, h * DH:(h + 1) * DH]
                    s = lax.dot_general(
                        qbh, kbh, (((1,), (1,)), ((), ())),
                        preferred_element_type=jnp.float32,
                    ) * 0.125
                    s = jnp.where(mask, s, NEG)
                    m = jnp.max(s, axis=1)
                    w = jnp.exp(s - m[:, None])
                    l = jnp.sum(w, axis=1)
                    o = jnp.dot(w, vbh, preferred_element_type=jnp.float32)
                    o_buf[0, bh] = o
                    ml_buf[0, bh] = m
                    ml_buf[0, BH + bh] = l

            partner = 1 - my_pos
            rd_o = pltpu.make_async_remote_copy(
                src_ref=o_buf.at[0], dst_ref=o_buf.at[1],
                send_sem=ex_send.at[0], recv_sem=ex_recv.at[0],
                device_id=(partner,), device_id_type=pl.DeviceIdType.MESH,
            )
            rd_ml = pltpu.make_async_remote_copy(
                src_ref=ml_buf.at[0], dst_ref=ml_buf.at[1],
                send_sem=ex_send.at[1], recv_sem=ex_recv.at[1],
                device_id=(partner,), device_id_type=pl.DeviceIdType.MESH,
            )
            rd_o.start()
            rd_ml.start()
            rd_o.wait()
            rd_ml.wait()

            m0 = ml_buf[0, 0:BH, :]
            l0 = ml_buf[0, BH:2 * BH, :]
            m1 = ml_buf[1, 0:BH, :]
            l1 = ml_buf[1, BH:2 * BH, :]
            m_new = jnp.maximum(m0, m1)
            fa = jnp.exp(m0 - m_new)
            fb = jnp.exp(m1 - m_new)
            o_run = fa[:, :, None] * o_buf[0] + fb[:, :, None] * o_buf[1]
            l_run = fa * l0 + fb * l1
            ctx = o_run / l_run[:, :, None]
            ctx_buf[...] = ctx

            rd_cx = pltpu.make_async_remote_copy(
                src_ref=ctx_buf, dst_ref=ctx_buf,
                send_sem=cx_send, recv_sem=cx_recv,
                device_id=(peer,), device_id_type=pl.DeviceIdType.MESH,
            )
            rd_cx.start()
            for b in range(B):
                ctx_b = jnp.concatenate(
                    [ctx[b * HQ + h] for h in range(HQ)], axis=1
                )
                out_ref[b] = jnp.dot(
                    ctx_b, wo_ref[...], preferred_element_type=jnp.float32
                )
            rd_cx.wait_send()

        @pl.when(jnp.logical_not(is01))
        def _consumer():
            rd_cx = pltpu.make_async_remote_copy(
                src_ref=ctx_buf, dst_ref=ctx_buf,
                send_sem=cx_send, recv_sem=cx_recv,
                device_id=(peer,), device_id_type=pl.DeviceIdType.MESH,
            )
            rd_cx.wait_recv()
            ctx = ctx_buf[...]
            for b in range(B):
                ctx_b = jnp.concatenate(
                    [ctx[b * HQ + h] for h in range(HQ)], axis=1
                )
                out_ref[b] = jnp.dot(
                    ctx_b, wo_ref[...], preferred_element_type=jnp.float32
                )

    return pl.pallas_call(
        body,
        out_shape=jax.ShapeDtypeStruct((B, SQ, 512), jnp.float32),
        in_specs=[pl.BlockSpec(memory_space=pltpu.VMEM)] * 5,
        out_specs=pl.BlockSpec(memory_space=pltpu.VMEM),
        scratch_shapes=[
            pltpu.VMEM((2, BH, SQ, DH), jnp.float32),
            pltpu.VMEM((2, 2 * BH, SQ), jnp.float32),
            pltpu.VMEM((BH, SQ, DH), jnp.float32),
            pltpu.SemaphoreType.DMA((2,)),
            pltpu.SemaphoreType.DMA((2,)),
            pltpu.SemaphoreType.DMA,
            pltpu.SemaphoreType.DMA,
        ],
        compiler_params=pltpu.CompilerParams(collective_id=0),
    )(x, Wq, K2, V2, Wo)


# device time: 25084 ns/iter; 2.0850x vs baseline; 1.4417x over previous
import jax
import jax.numpy as jnp
from jax import lax
from jax.experimental import pallas as pl
from jax.experimental.pallas import tpu as pltpu

N_DEV = 4
B = 2
SQ = 256
SKV = 256
HQ = 4
DH = 64
D = HQ * DH
WINDOW = 128
NEG = -1e9
ROWS = B * SQ


def kernel(x, Wq, K_ext, V_ext, Wo):
    K2 = K_ext.reshape(B, SKV, D)
    V2 = V_ext.reshape(B, SKV, D)

    def body(x_ref, wq_ref, k_ref, v_ref, wo_ref, out_ref,
             o_buf, l_buf, ctx_buf, ex_send, ex_recv, cx_send, cx_recv):
        my_pos = lax.axis_index("i")
        left = lax.rem(my_pos + (N_DEV - 1), N_DEV)
        right = lax.rem(my_pos + 1, N_DEV)
        peer = (N_DEV - 1) - my_pos
        is01 = my_pos <= 1

        barrier_sem = pltpu.get_barrier_semaphore()
        for nbr in (left, right):
            pl.semaphore_signal(
                barrier_sem, inc=1,
                device_id=(nbr,), device_id_type=pl.DeviceIdType.MESH,
            )
        pl.semaphore_wait(barrier_sem, 2)

        @pl.when(is01)
        def _producer():
            kv_off = my_pos * SKV
            qi = lax.broadcasted_iota(jnp.int32, (SQ, SKV), 0)
            kj = lax.broadcasted_iota(jnp.int32, (SQ, SKV), 1) + kv_off
            bias = jnp.where(jnp.abs(qi - kj) <= WINDOW, 0.0, NEG)

            x2d = x_ref[...].reshape(ROWS, 512)
            qall = jnp.dot(x2d, wq_ref[...],
                           preferred_element_type=jnp.float32)
            for b in range(B):
                kb = k_ref[b]
                vb = v_ref[b]
                for h in range(HQ):
                    hs = slice(h * DH, (h + 1) * DH)
                    qbh = qall[b * SQ:(b + 1) * SQ, hs]
                    s = lax.dot_general(
                        qbh, kb[:, hs], (((1,), (1,)), ((), ())),
                        preferred_element_type=jnp.float32,
                    )
                    w = jnp.exp(s * 0.125 + bias)
                    l = jnp.sum(w, axis=1, keepdims=True)
                    o = jnp.dot(w, vb[:, hs],
                                preferred_element_type=jnp.float32)
                    o_buf[0, b * SQ:(b + 1) * SQ, hs] = o
                    l_buf[0, b * SQ:(b + 1) * SQ, h:h + 1] = l

            partner = 1 - my_pos
            rd_o = pltpu.make_async_remote_copy(
                src_ref=o_buf.at[0], dst_ref=o_buf.at[1],
                send_sem=ex_send.at[0], recv_sem=ex_recv.at[0],
                device_id=(partner,), device_id_type=pl.DeviceIdType.MESH,
            )
            rd_l = pltpu.make_async_remote_copy(
                src_ref=l_buf.at[0], dst_ref=l_buf.at[1],
                send_sem=ex_send.at[1], recv_sem=ex_recv.at[1],
                device_id=(partner,), device_id_type=pl.DeviceIdType.MESH,
            )
            rd_o.start()
            rd_l.start()
            rd_o.wait()
            rd_l.wait()

            o_sum = o_buf[0] + o_buf[1]
            l_sum = l_buf[0] + l_buf[1]
            for h in range(HQ):
                hs = slice(h * DH, (h + 1) * DH)
                ctx_buf[:, hs] = o_sum[:, hs] / l_sum[:, h:h + 1]

            rd_cx = pltpu.make_async_remote_copy(
                src_ref=ctx_buf, dst_ref=ctx_buf,
                send_sem=cx_send, recv_sem=cx_recv,
                device_id=(peer,), device_id_type=pl.DeviceIdType.MESH,
            )
            rd_cx.start()
            out = jnp.dot(ctx_buf[...], wo_ref[...],
                          preferred_element_type=jnp.float32)
            out_ref[...] = out.reshape(B, SQ, 512)
            rd_cx.wait_send()

        @pl.when(jnp.logical_not(is01))
        def _consumer():
            rd_cx = pltpu.make_async_remote_copy(
                src_ref=ctx_buf, dst_ref=ctx_buf,
                send_sem=cx_send, recv_sem=cx_recv,
                device_id=(peer,), device_id_type=pl.DeviceIdType.MESH,
            )
            rd_cx.wait_recv()
            out = jnp.dot(ctx_buf[...], wo_ref[...],
                          preferred_element_type=jnp.float32)
            out_ref[...] = out.reshape(B, SQ, 512)

    return pl.pallas_call(
        body,
        out_shape=jax.ShapeDtypeStruct((B, SQ, 512), jnp.float32),
        in_specs=[pl.BlockSpec(memory_space=pltpu.VMEM)] * 5,
        out_specs=pl.BlockSpec(memory_space=pltpu.VMEM),
        scratch_shapes=[
            pltpu.VMEM((2, ROWS, D), jnp.float32),
            pltpu.VMEM((2, ROWS, HQ), jnp.float32),
            pltpu.VMEM((ROWS, D), jnp.float32),
            pltpu.SemaphoreType.DMA((2,)),
            pltpu.SemaphoreType.DMA((2,)),
            pltpu.SemaphoreType.DMA,
            pltpu.SemaphoreType.DMA,
        ],
        compiler_params=pltpu.CompilerParams(collective_id=0),
    )(x, Wq, K2, V2, Wo)


# device time: 17839 ns/iter; 2.9317x vs baseline; 1.4061x over previous
import jax
import jax.numpy as jnp
from jax import lax
from jax.experimental import pallas as pl
from jax.experimental.pallas import tpu as pltpu

N_DEV = 4
B = 2
SQ = 256
SKV = 256
HQ = 4
DH = 64
D = HQ * DH
HALF = 128
WINDOW = 128
NEG = -1e9
DL = D + 8


def kernel(x, Wq, K_ext, V_ext, Wo):
    K2 = K_ext.reshape(B, SKV, D)
    V2 = V_ext.reshape(B, SKV, D)

    def body(x_ref, wq_ref, k_ref, v_ref, wo_ref, out_ref,
             o_buf, ctx_buf, ex_send, ex_recv, cx_send, cx_recv):
        my_pos = lax.axis_index("i")
        left = lax.rem(my_pos + (N_DEV - 1), N_DEV)
        right = lax.rem(my_pos + 1, N_DEV)
        is01 = my_pos <= 1

        barrier_sem = pltpu.get_barrier_semaphore()
        for nbr in (left, right):
            pl.semaphore_signal(
                barrier_sem, inc=1,
                device_id=(nbr,), device_id_type=pl.DeviceIdType.MESH,
            )
        pl.semaphore_wait(barrier_sem, 2)

        def proj(rows):
            n = rows.stop - rows.start
            c = ctx_buf[:, rows, :].reshape(B * n, D)
            o = jnp.dot(c, wo_ref[...], preferred_element_type=jnp.float32)
            out_ref[:, rows, :] = o.reshape(B, n, 512)

        @pl.when(is01)
        def _producer():
            partner = 1 - my_pos

            @pl.when(my_pos == 0)
            def _p0_partial():
                qi = lax.broadcasted_iota(jnp.int32, (SQ, SKV), 0)
                kj = lax.broadcasted_iota(jnp.int32, (SQ, SKV), 1)
                bias = jnp.where(jnp.abs(qi - kj) <= WINDOW, 0.0, NEG)
                x2d = x_ref[...].reshape(B * SQ, 512)
                qall = jnp.dot(x2d, wq_ref[...],
                               preferred_element_type=jnp.float32)
                for b in range(B):
                    kb = k_ref[b]
                    vb = v_ref[b]
                    for h in range(HQ):
                        hs = slice(h * DH, (h + 1) * DH)
                        qbh = qall[b * SQ:(b + 1) * SQ, hs]
                        s = lax.dot_general(
                            qbh, kb[:, hs], (((1,), (1,)), ((), ())),
                            preferred_element_type=jnp.float32,
                        )
                        w = jnp.exp(s * 0.125 + bias)
                        l = jnp.sum(w, axis=1, keepdims=True)
                        o = jnp.dot(w, vb[:, hs],
                                    preferred_element_type=jnp.float32)
                        o_buf[0, b, :, hs] = o
                        o_buf[0, b, :, D + h:D + h + 1] = l

            @pl.when(my_pos == 1)
            def _p1_partial():
                ti = lax.broadcasted_iota(jnp.int32, (HALF, HALF), 0)
                tj = lax.broadcasted_iota(jnp.int32, (HALF, HALF), 1)
                bias = jnp.where(tj <= ti, 0.0, NEG)
                xhi = x_ref[:, HALF:, :].reshape(B * HALF, 512)
                qhi = jnp.dot(xhi, wq_ref[...],
                              preferred_element_type=jnp.float32)
                for b in range(B):
                    kb = k_ref[b]
                    vb = v_ref[b]
                    for h in range(HQ):
                        hs = slice(h * DH, (h + 1) * DH)
                        qbh = qhi[b * HALF:(b + 1) * HALF, hs]
                        s = lax.dot_general(
                            qbh, kb[0:HALF, hs], (((1,), (1,)), ((), ())),
                            preferred_element_type=jnp.float32,
                        )
                        w = jnp.exp(s * 0.125 + bias)
                        l = jnp.sum(w, axis=1, keepdims=True)
                        o = jnp.dot(w, vb[0:HALF, hs],
                                    preferred_element_type=jnp.float32)
                        o_buf[0, b, HALF:, hs] = o
                        o_buf[0, b, HALF:, D + h:D + h + 1] = l

            rd_ex = pltpu.make_async_remote_copy(
                src_ref=o_buf.at[0, :, pl.ds(HALF, HALF), :],
                dst_ref=o_buf.at[1, :, pl.ds(HALF, HALF), :],
                send_sem=ex_send, recv_sem=ex_recv,
                device_id=(partner,), device_id_type=pl.DeviceIdType.MESH,
            )
            rd_ex.start()

            rd_lo3 = pltpu.make_async_remote_copy(
                src_ref=ctx_buf.at[:, pl.ds(0, HALF), :],
                dst_ref=ctx_buf.at[:, pl.ds(0, HALF), :],
                send_sem=cx_send.at[0], recv_sem=cx_recv.at[0],
                device_id=(3,), device_id_type=pl.DeviceIdType.MESH,
            )
            rd_lo1 = pltpu.make_async_remote_copy(
                src_ref=ctx_buf.at[:, pl.ds(0, HALF), :],
                dst_ref=ctx_buf.at[:, pl.ds(0, HALF), :],
                send_sem=cx_send.at[1], recv_sem=cx_recv.at[0],
                device_id=(1,), device_id_type=pl.DeviceIdType.MESH,
            )

            @pl.when(my_pos == 0)
            def _p0_ctx_lo():
                for h in range(HQ):
                    hs = slice(h * DH, (h + 1) * DH)
                    ctx_buf[:, 0:HALF, hs] = (
                        o_buf[0, :, 0:HALF, hs]
                        / o_buf[0, :, 0:HALF, D + h:D + h + 1]
                    )
                rd_lo3.start()
                rd_lo1.start()

            rd_ex.wait()

            p_sum = o_buf[0, :, HALF:, :] + o_buf[1, :, HALF:, :]
            for h in range(HQ):
                hs = slice(h * DH, (h + 1) * DH)
                ctx_buf[:, HALF:, hs] = (
                    p_sum[:, :, hs] / p_sum[:, :, D + h:D + h + 1]
                )

            @pl.when(my_pos == 0)
            def _p0_finish():
                rd_hi3 = pltpu.make_async_remote_copy(
                    src_ref=ctx_buf.at[:, pl.ds(HALF, HALF), :],
                    dst_ref=ctx_buf.at[:, pl.ds(HALF, HALF), :],
                    send_sem=cx_send.at[2], recv_sem=cx_recv.at[1],
                    device_id=(3,), device_id_type=pl.DeviceIdType.MESH,
                )
                rd_hi3.start()
                proj(slice(0, SQ))
                rd_lo3.wait_send()
                rd_lo1.wait_send()
                rd_hi3.wait_send()

            @pl.when(my_pos == 1)
            def _p1_finish():
                rd_hi2 = pltpu.make_async_remote_copy(
                    src_ref=ctx_buf.at[:, pl.ds(HALF, HALF), :],
                    dst_ref=ctx_buf.at[:, pl.ds(HALF, HALF), :],
                    send_sem=cx_send.at[0], recv_sem=cx_recv.at[1],
                    device_id=(2,), device_id_type=pl.DeviceIdType.MESH,
                )
                rd_hi2.start()
                proj(slice(HALF, SQ))
                rd_lo1.wait_recv()
                proj(slice(0, HALF))
                rd_hi2.wait_send()

        @pl.when(jnp.logical_not(is01))
        def _consumer():
            rd_lo = pltpu.make_async_remote_copy(
                src_ref=ctx_buf.at[:, pl.ds(0, HALF), :],
                dst_ref=ctx_buf.at[:, pl.ds(0, HALF), :],
                send_sem=cx_send.at[1], recv_sem=cx_recv.at[0],
                device_id=(left,), device_id_type=pl.DeviceIdType.MESH,
            )
            rd_hi = pltpu.make_async_remote_copy(
                src_ref=ctx_buf.at[:, pl.ds(HALF, HALF), :],
                dst_ref=ctx_buf.at[:, pl.ds(HALF, HALF), :],
                send_sem=cx_send.at[2], recv_sem=cx_recv.at[1],
                device_id=(left,), device_id_type=pl.DeviceIdType.MESH,
            )

            @pl.when(my_pos == 3)
            def _c3():
                rd_lo.wait_recv()
                rd_fwd = pltpu.make_async_remote_copy(
                    src_ref=ctx_buf.at[:, pl.ds(0, HALF), :],
                    dst_ref=ctx_buf.at[:, pl.ds(0, HALF), :],
                    send_sem=cx_send.at[0], recv_sem=cx_recv.at[0],
                    device_id=(2,), device_id_type=pl.DeviceIdType.MESH,
                )
                rd_fwd.start()
                proj(slice(0, HALF))
                rd_hi.wait_recv()
                proj(slice(HALF, SQ))
                rd_fwd.wait_send()

            @pl.when(my_pos == 2)
            def _c2():
                rd_lo.wait_recv()
                rd_hi.wait_recv()
                proj(slice(0, SQ))

    return pl.pallas_call(
        body,
        out_shape=jax.ShapeDtypeStruct((B, SQ, 512), jnp.float32),
        in_specs=[pl.BlockSpec(memory_space=pltpu.VMEM)] * 5,
        out_specs=pl.BlockSpec(memory_space=pltpu.VMEM),
        scratch_shapes=[
            pltpu.VMEM((2, B, SQ, DL), jnp.float32),
            pltpu.VMEM((B, SQ, D), jnp.float32),
            pltpu.SemaphoreType.DMA,
            pltpu.SemaphoreType.DMA,
            pltpu.SemaphoreType.DMA((3,)),
            pltpu.SemaphoreType.DMA((2,)),
        ],
        compiler_params=pltpu.CompilerParams(collective_id=0),
    )(x, Wq, K2, V2, Wo)


# device time: 14376 ns/iter; 3.6379x vs baseline; 1.2409x over previous
import jax
import jax.numpy as jnp
from jax import lax
from jax.experimental import pallas as pl
from jax.experimental.pallas import tpu as pltpu

N_DEV = 4
B = 2
SQ = 256
SKV = 256
HQ = 4
DH = 64
D = HQ * DH
HALF = 128
NEG = -1e9
DL = D + 8
BF = jnp.bfloat16


def kernel(x, Wq, K_ext, V_ext, Wo):
    K2 = K_ext.reshape(B, SKV, D)
    V2 = V_ext.reshape(B, SKV, D)

    def body(x_ref, wq_ref, k_ref, v_ref, wo_ref, out_ref,
             o_buf, ctx_buf, ex_send, ex_recv, cx_send, cx_recv):
        my_pos = lax.axis_index("i")
        left = lax.rem(my_pos + (N_DEV - 1), N_DEV)
        right = lax.rem(my_pos + 1, N_DEV)
        is01 = my_pos <= 1

        barrier_sem = pltpu.get_barrier_semaphore()
        for nbr in (left, right):
            pl.semaphore_signal(
                barrier_sem, inc=1,
                device_id=(nbr,), device_id_type=pl.DeviceIdType.MESH,
            )
        pl.semaphore_wait(barrier_sem, 2)

        def proj(rows):
            n = rows.stop - rows.start
            c = ctx_buf[:, rows, :].reshape(B * n, D)
            o = jnp.dot(c, wo_ref[...].astype(BF),
                        preferred_element_type=jnp.float32)
            out_ref[:, rows, :] = o.reshape(B, n, 512)

        def attn_block(qrows, kb, vb, h, bias):
            hs = slice(h * DH, (h + 1) * DH)
            s = lax.dot_general(
                qrows, kb[:, hs], (((1,), (1,)), ((), ())),
                preferred_element_type=jnp.float32,
            )
            w = jnp.exp(s * 0.125 + bias)
            l = jnp.sum(w, axis=1, keepdims=True)
            o = jnp.dot(w.astype(BF), vb[:, hs],
                        preferred_element_type=jnp.float32)
            return o, l

        @pl.when(is01)
        def _producer():
            partner = 1 - my_pos

            @pl.when(my_pos == 0)
            def _p0_hi():
                ti = lax.broadcasted_iota(jnp.int32, (HALF, SKV), 0)
                tj = lax.broadcasted_iota(jnp.int32, (HALF, SKV), 1)
                bias = jnp.where(tj >= ti, 0.0, NEG)
                xhi = x_ref[:, HALF:, :].reshape(B * HALF, 512).astype(BF)
                qhi = jnp.dot(xhi, wq_ref[...].astype(BF),
                              preferred_element_type=jnp.float32).astype(BF)
                for b in range(B):
                    kb = k_ref[b].astype(BF)
                    vb = v_ref[b].astype(BF)
                    for h in range(HQ):
                        hs = slice(h * DH, (h + 1) * DH)
                        o, l = attn_block(
                            qhi[b * HALF:(b + 1) * HALF, hs], kb, vb, h, bias)
                        o_buf[0, b, HALF:, hs] = o.astype(BF)
                        o_buf[0, b, HALF:, D + h:D + h + 1] = l.astype(BF)

            @pl.when(my_pos == 1)
            def _p1_partial():
                ti = lax.broadcasted_iota(jnp.int32, (HALF, HALF), 0)
                tj = lax.broadcasted_iota(jnp.int32, (HALF, HALF), 1)
                bias = jnp.where(tj <= ti, 0.0, NEG)
                xhi = x_ref[:, HALF:, :].reshape(B * HALF, 512).astype(BF)
                qhi = jnp.dot(xhi, wq_ref[...].astype(BF),
                              preferred_element_type=jnp.float32).astype(BF)
                for b in range(B):
                    kb = k_ref[b, 0:HALF, :].astype(BF)
                    vb = v_ref[b, 0:HALF, :].astype(BF)
                    for h in range(HQ):
                        hs = slice(h * DH, (h + 1) * DH)
                        o, l = attn_block(
                            qhi[b * HALF:(b + 1) * HALF, hs], kb, vb, h, bias)
                        o_buf[0, b, HALF:, hs] = o.astype(BF)
                        o_buf[0, b, HALF:, D + h:D + h + 1] = l.astype(BF)

            rd_ex = pltpu.make_async_remote_copy(
                src_ref=o_buf.at[0, :, pl.ds(HALF, HALF), :],
                dst_ref=o_buf.at[1, :, pl.ds(HALF, HALF), :],
                send_sem=ex_send, recv_sem=ex_recv,
                device_id=(partner,), device_id_type=pl.DeviceIdType.MESH,
            )
            rd_ex.start()

            rd_lo3 = pltpu.make_async_remote_copy(
                src_ref=ctx_buf.at[:, pl.ds(0, HALF), :],
                dst_ref=ctx_buf.at[:, pl.ds(0, HALF), :],
                send_sem=cx_send.at[0], recv_sem=cx_recv.at[0],
                device_id=(3,), device_id_type=pl.DeviceIdType.MESH,
            )
            rd_lo1 = pltpu.make_async_remote_copy(
                src_ref=ctx_buf.at[:, pl.ds(0, HALF), :],
                dst_ref=ctx_buf.at[:, pl.ds(0, HALF), :],
                send_sem=cx_send.at[1], recv_sem=cx_recv.at[0],
                device_id=(1,), device_id_type=pl.DeviceIdType.MESH,
            )

            @pl.when(my_pos == 0)
            def _p0_lo():
                ti = lax.broadcasted_iota(jnp.int32, (HALF, SKV), 0)
                tj = lax.broadcasted_iota(jnp.int32, (HALF, SKV), 1)
                bias = jnp.where(tj - ti <= 128, 0.0, NEG)
                xlo = x_ref[:, 0:HALF, :].reshape(B * HALF, 512).astype(BF)
                qlo = jnp.dot(xlo, wq_ref[...].astype(BF),
                              preferred_element_type=jnp.float32).astype(BF)
                for b in range(B):
                    kb = k_ref[b].astype(BF)
                    vb = v_ref[b].astype(BF)
                    for h in range(HQ):
                        hs = slice(h * DH, (h + 1) * DH)
                        o, l = attn_block(
                            qlo[b * HALF:(b + 1) * HALF, hs], kb, vb, h, bias)
                        ctx_buf[b, 0:HALF, hs] = (o / l).astype(BF)
                rd_lo3.start()
                rd_lo1.start()

            rd_ex.wait()

            p_sum = (o_buf[0, :, HALF:, :].astype(jnp.float32)
                     + o_buf[1, :, HALF:, :].astype(jnp.float32))
            for h in range(HQ):
                hs = slice(h * DH, (h + 1) * DH)
                ctx_buf[:, HALF:, hs] = (
                    p_sum[:, :, hs] / p_sum[:, :, D + h:D + h + 1]
                ).astype(BF)

            @pl.when(my_pos == 0)
            def _p0_finish():
                rd_hi3 = pltpu.make_async_remote_copy(
                    src_ref=ctx_buf.at[:, pl.ds(HALF, HALF), :],
                    dst_ref=ctx_buf.at[:, pl.ds(HALF, HALF), :],
                    send_sem=cx_send.at[2], recv_sem=cx_recv.at[1],
                    device_id=(3,), device_id_type=pl.DeviceIdType.MESH,
                )
                rd_hi3.start()
                proj(slice(0, SQ))
                rd_lo3.wait_send()
                rd_lo1.wait_send()
                rd_hi3.wait_send()

            @pl.when(my_pos == 1)
            def _p1_finish():
                rd_hi2 = pltpu.make_async_remote_copy(
                    src_ref=ctx_buf.at[:, pl.ds(HALF, HALF), :],
                    dst_ref=ctx_buf.at[:, pl.ds(HALF, HALF), :],
                    send_sem=cx_send.at[0], recv_sem=cx_recv.at[1],
                    device_id=(2,), device_id_type=pl.DeviceIdType.MESH,
                )
                rd_hi2.start()
                proj(slice(HALF, SQ))
                rd_lo1.wait_recv()
                proj(slice(0, HALF))
                rd_hi2.wait_send()

        @pl.when(jnp.logical_not(is01))
        def _consumer():
            rd_lo = pltpu.make_async_remote_copy(
                src_ref=ctx_buf.at[:, pl.ds(0, HALF), :],
                dst_ref=ctx_buf.at[:, pl.ds(0, HALF), :],
                send_sem=cx_send.at[1], recv_sem=cx_recv.at[0],
                device_id=(left,), device_id_type=pl.DeviceIdType.MESH,
            )
            rd_hi = pltpu.make_async_remote_copy(
                src_ref=ctx_buf.at[:, pl.ds(HALF, HALF), :],
                dst_ref=ctx_buf.at[:, pl.ds(HALF, HALF), :],
                send_sem=cx_send.at[2], recv_sem=cx_recv.at[1],
                device_id=(left,), device_id_type=pl.DeviceIdType.MESH,
            )

            @pl.when(my_pos == 3)
            def _c3():
                rd_lo.wait_recv()
                rd_fwd = pltpu.make_async_remote_copy(
                    src_ref=ctx_buf.at[:, pl.ds(0, HALF), :],
                    dst_ref=ctx_buf.at[:, pl.ds(0, HALF), :],
                    send_sem=cx_send.at[0], recv_sem=cx_recv.at[0],
                    device_id=(2,), device_id_type=pl.DeviceIdType.MESH,
                )
                rd_fwd.start()
                proj(slice(0, HALF))
                rd_hi.wait_recv()
                proj(slice(HALF, SQ))
                rd_fwd.wait_send()

            @pl.when(my_pos == 2)
            def _c2():
                rd_lo.wait_recv()
                proj(slice(0, HALF))
                rd_hi.wait_recv()
                proj(slice(HALF, SQ))

    return pl.pallas_call(
        body,
        out_shape=jax.ShapeDtypeStruct((B, SQ, 512), jnp.float32),
        in_specs=[pl.BlockSpec(memory_space=pltpu.VMEM)] * 5,
        out_specs=pl.BlockSpec(memory_space=pltpu.VMEM),
        scratch_shapes=[
            pltpu.VMEM((2, B, SQ, DL), BF),
            pltpu.VMEM((B, SQ, D), BF),
            pltpu.SemaphoreType.DMA,
            pltpu.SemaphoreType.DMA,
            pltpu.SemaphoreType.DMA((3,)),
            pltpu.SemaphoreType.DMA((2,)),
        ],
        compiler_params=pltpu.CompilerParams(collective_id=0),
    )(x, Wq, K2, V2, Wo)


# device time: 13640 ns/iter; 3.8342x vs baseline; 1.0540x over previous
import jax
import jax.numpy as jnp
from jax import lax
from jax.experimental import pallas as pl
from jax.experimental.pallas import tpu as pltpu

N_DEV = 4
B = 2
SQ = 256
SKV = 256
HQ = 4
DH = 64
D = HQ * DH
HALF = 128
NEG = -1e9
DL = D + 8
BF = jnp.bfloat16


def kernel(x, Wq, K_ext, V_ext, Wo):
    K2 = K_ext.reshape(B, SKV, D).astype(BF)
    V2 = V_ext.reshape(B, SKV, D).astype(BF)

    def body(x_ref, wq_ref, k_ref, v_ref, wo_ref, out_ref,
             o_buf, ctx_buf, ex_send, ex_recv, cx_send, cx_recv):
        my_pos = lax.axis_index("i")
        left = lax.rem(my_pos + (N_DEV - 1), N_DEV)
        right = lax.rem(my_pos + 1, N_DEV)
        is01 = my_pos <= 1

        barrier_sem = pltpu.get_barrier_semaphore()
        for nbr in (left, right):
            pl.semaphore_signal(
                barrier_sem, inc=1,
                device_id=(nbr,), device_id_type=pl.DeviceIdType.MESH,
            )
        pl.semaphore_wait(barrier_sem, 2)

        def proj(rows, bs=None):
            n = rows.stop - rows.start
            if bs is None:
                c = ctx_buf[:, rows, :].reshape(B * n, D)
                o = jnp.dot(c, wo_ref[...].astype(BF),
                            preferred_element_type=jnp.float32)
                out_ref[:, rows, :] = o.reshape(B, n, 512)
            else:
                c = ctx_buf[bs, rows, :]
                o = jnp.dot(c, wo_ref[...].astype(BF),
                            preferred_element_type=jnp.float32)
                out_ref[bs, rows, :] = o

        def attn_block(qrows, kbh, vbh, bias):
            s = lax.dot_general(
                qrows, kbh, (((1,), (1,)), ((), ())),
                preferred_element_type=jnp.float32,
            )
            w = jnp.exp((s + bias).astype(BF))
            l = jnp.sum(w, axis=1, keepdims=True, dtype=jnp.float32)
            o = jnp.dot(w, vbh, preferred_element_type=jnp.float32)
            return o, l

        def combine_hi(b):
            p = o_buf[0, b, HALF:, :] + o_buf[1, b, HALF:, :]
            for h in range(HQ):
                hs = slice(h * DH, (h + 1) * DH)
                ctx_buf[b, HALF:, hs] = p[:, hs] / p[:, D + h:D + h + 1]

        @pl.when(is01)
        def _producer():
            partner = 1 - my_pos
            rd_ex = [
                pltpu.make_async_remote_copy(
                    src_ref=o_buf.at[0, b, pl.ds(HALF, HALF), :],
                    dst_ref=o_buf.at[1, b, pl.ds(HALF, HALF), :],
                    send_sem=ex_send.at[b], recv_sem=ex_recv.at[b],
                    device_id=(partner,),
                    device_id_type=pl.DeviceIdType.MESH,
                )
                for b in range(B)
            ]
            rd_lo3 = pltpu.make_async_remote_copy(
                src_ref=ctx_buf.at[:, pl.ds(0, HALF), :],
                dst_ref=ctx_buf.at[:, pl.ds(0, HALF), :],
                send_sem=cx_send.at[0], recv_sem=cx_recv.at[0],
                device_id=(3,), device_id_type=pl.DeviceIdType.MESH,
            )
            rd_lo1 = pltpu.make_async_remote_copy(
                src_ref=ctx_buf.at[:, pl.ds(0, HALF), :],
                dst_ref=ctx_buf.at[:, pl.ds(0, HALF), :],
                send_sem=cx_send.at[1], recv_sem=cx_recv.at[0],
                device_id=(1,), device_id_type=pl.DeviceIdType.MESH,
            )

            @pl.when(my_pos == 0)
            def _p0():
                ti = lax.broadcasted_iota(jnp.int32, (HALF, SKV), 0)
                tj = lax.broadcasted_iota(jnp.int32, (HALF, SKV), 1)
                bias = jnp.where(tj >= ti, 0.0, NEG)
                xhi = x_ref[:, HALF:, :].reshape(B * HALF, 512).astype(BF)
                qhi = (jnp.dot(xhi, wq_ref[...].astype(BF),
                               preferred_element_type=jnp.float32)
                       * 0.125).astype(BF)
                for b in range(B):
                    kb = k_ref[b]
                    vb = v_ref[b]
                    for h in range(HQ):
                        hs = slice(h * DH, (h + 1) * DH)
                        o, l = attn_block(
                            qhi[b * HALF:(b + 1) * HALF, hs],
                            kb[:, hs], vb[:, hs], bias)
                        o_buf[0, b, HALF:, hs] = o.astype(BF)
                        o_buf[0, b, HALF:, D + h:D + h + 1] = l.astype(BF)
                    rd_ex[b].start()

                bias_lo = jnp.where(tj - ti <= 128, 0.0, NEG)
                xlo = x_ref[:, 0:HALF, :].reshape(B * HALF, 512).astype(BF)
                qlo = (jnp.dot(xlo, wq_ref[...].astype(BF),
                               preferred_element_type=jnp.float32)
                       * 0.125).astype(BF)
                for b in range(B):
                    kb = k_ref[b]
                    vb = v_ref[b]
                    for h in range(HQ):
                        hs = slice(h * DH, (h + 1) * DH)
                        o, l = attn_block(
                            qlo[b * HALF:(b + 1) * HALF, hs],
                            kb[:, hs], vb[:, hs], bias_lo)
                        ctx_buf[b, 0:HALF, hs] = (o / l).astype(BF)
                rd_lo3.start()
                rd_lo1.start()

                rd_ex[0].wait()
                combine_hi(0)
                rd_ex[1].wait()
                combine_hi(1)
                rd_hi3 = pltpu.make_async_remote_copy(
                    src_ref=ctx_buf.at[:, pl.ds(HALF, HALF), :],
                    dst_ref=ctx_buf.at[:, pl.ds(HALF, HALF), :],
                    send_sem=cx_send.at[2], recv_sem=cx_recv.at[1],
                    device_id=(3,), device_id_type=pl.DeviceIdType.MESH,
                )
                rd_hi3.start()
                proj(slice(0, SQ))
                rd_lo3.wait_send()
                rd_lo1.wait_send()
                rd_hi3.wait_send()

            @pl.when(my_pos == 1)
            def _p1():
                ti = lax.broadcasted_iota(jnp.int32, (HALF, HALF), 0)
                tj = lax.broadcasted_iota(jnp.int32, (HALF, HALF), 1)
                bias = jnp.where(tj <= ti, 0.0, NEG)
                xhi = x_ref[:, HALF:, :].reshape(B * HALF, 512).astype(BF)
                qhi = (jnp.dot(xhi, wq_ref[...].astype(BF),
                               preferred_element_type=jnp.float32)
                       * 0.125).astype(BF)
                for b in range(B):
                    kb = k_ref[b, 0:HALF, :]
                    vb = v_ref[b, 0:HALF, :]
                    for h in range(HQ):
                        hs = slice(h * DH, (h + 1) * DH)
                        o, l = attn_block(
                            qhi[b * HALF:(b + 1) * HALF, hs],
                            kb[:, hs], vb[:, hs], bias)
                        o_buf[0, b, HALF:, hs] = o.astype(BF)
                        o_buf[0, b, HALF:, D + h:D + h + 1] = l.astype(BF)
                    rd_ex[b].start()

                rd_hi2 = [
                    pltpu.make_async_remote_copy(
                        src_ref=ctx_buf.at[b, pl.ds(HALF, HALF), :],
                        dst_ref=ctx_buf.at[b, pl.ds(HALF, HALF), :],
                        send_sem=cx_send.at[2 + b],
                        recv_sem=cx_recv.at[1 + b],
                        device_id=(2,), device_id_type=pl.DeviceIdType.MESH,
                    )
                    for b in range(B)
                ]
                for b in range(B):
                    rd_ex[b].wait()
                    combine_hi(b)
                    rd_hi2[b].start()
                    proj(slice(HALF, SQ), bs=b)
                rd_lo1.wait_recv()
                proj(slice(0, HALF))
                rd_hi2[0].wait_send()
                rd_hi2[1].wait_send()

        @pl.when(jnp.logical_not(is01))
        def _consumer():
            rd_lo = pltpu.make_async_remote_copy(
                src_ref=ctx_buf.at[:, pl.ds(0, HALF), :],
                dst_ref=ctx_buf.at[:, pl.ds(0, HALF), :],
                send_sem=cx_send.at[1], recv_sem=cx_recv.at[0],
                device_id=(left,), device_id_type=pl.DeviceIdType.MESH,
            )

            @pl.when(my_pos == 3)
            def _c3():
                rd_hi = pltpu.make_async_remote_copy(
                    src_ref=ctx_buf.at[:, pl.ds(HALF, HALF), :],
                    dst_ref=ctx_buf.at[:, pl.ds(HALF, HALF), :],
                    send_sem=cx_send.at[2], recv_sem=cx_recv.at[1],
                    device_id=(0,), device_id_type=pl.DeviceIdType.MESH,
                )
                rd_lo.wait_recv()
                rd_fwd = pltpu.make_async_remote_copy(
                    src_ref=ctx_buf.at[:, pl.ds(0, HALF), :],
                    dst_ref=ctx_buf.at[:, pl.ds(0, HALF), :],
                    send_sem=cx_send.at[0], recv_sem=cx_recv.at[0],
                    device_id=(2,), device_id_type=pl.DeviceIdType.MESH,
                )
                rd_fwd.start()
                proj(slice(0, HALF))
                rd_hi.wait_recv()
                proj(slice(HALF, SQ))
                rd_fwd.wait_send()

            @pl.when(my_pos == 2)
            def _c2():
                rd_hi2 = [
                    pltpu.make_async_remote_copy(
                        src_ref=ctx_buf.at[b, pl.ds(HALF, HALF), :],
                        dst_ref=ctx_buf.at[b, pl.ds(HALF, HALF), :],
                        send_sem=cx_send.at[2 + b],
                        recv_sem=cx_recv.at[1 + b],
                        device_id=(1,), device_id_type=pl.DeviceIdType.MESH,
                    )
                    for b in range(B)
                ]
                rd_hi2[0].wait_recv()
                proj(slice(HALF, SQ), bs=0)
                rd_hi2[1].wait_recv()
                proj(slice(HALF, SQ), bs=1)
                rd_lo.wait_recv()
                proj(slice(0, HALF))

    return pl.pallas_call(
        body,
        out_shape=jax.ShapeDtypeStruct((B, SQ, 512), jnp.float32),
        in_specs=[pl.BlockSpec(memory_space=pltpu.VMEM)] * 5,
        out_specs=pl.BlockSpec(memory_space=pltpu.VMEM),
        scratch_shapes=[
            pltpu.VMEM((2, B, SQ, DL), BF),
            pltpu.VMEM((B, SQ, D), BF),
            pltpu.SemaphoreType.DMA((B,)),
            pltpu.SemaphoreType.DMA((B,)),
            pltpu.SemaphoreType.DMA((4,)),
            pltpu.SemaphoreType.DMA((3,)),
        ],
        compiler_params=pltpu.CompilerParams(collective_id=0),
    )(x, Wq, K2, V2, Wo)


# device time: 13625 ns/iter; 3.8385x vs baseline; 1.0011x over previous
import jax
import jax.numpy as jnp
from jax import lax
from jax.experimental import pallas as pl
from jax.experimental.pallas import tpu as pltpu

N_DEV = 4
B = 2
SQ = 256
SKV = 256
HQ = 4
DH = 64
D = HQ * DH
HALF = 128
NEG = -1e9
DL = D + 8
BF = jnp.bfloat16


def kernel(x, Wq, K_ext, V_ext, Wo):
    K2 = K_ext.reshape(B, SKV, D).astype(BF)
    V2 = V_ext.reshape(B, SKV, D).astype(BF)

    def body(x_ref, wq_ref, k_ref, v_ref, wo_ref, out_ref,
             o_buf, ctx_buf, ex_send, ex_recv, cx_send, cx_recv):
        my_pos = lax.axis_index("i")
        left = lax.rem(my_pos + (N_DEV - 1), N_DEV)
        right = lax.rem(my_pos + 1, N_DEV)
        is01 = my_pos <= 1

        barrier_sem = pltpu.get_barrier_semaphore()
        for nbr in (left, right):
            pl.semaphore_signal(
                barrier_sem, inc=1,
                device_id=(nbr,), device_id_type=pl.DeviceIdType.MESH,
            )
        pl.semaphore_wait(barrier_sem, 2)

        def proj(rows, bs=None):
            n = rows.stop - rows.start
            if bs is None:
                c = ctx_buf[:, rows, :].reshape(B * n, D)
                o = jnp.dot(c, wo_ref[...].astype(BF),
                            preferred_element_type=jnp.float32)
                out_ref[:, rows, :] = o.reshape(B, n, 512)
            else:
                c = ctx_buf[bs, rows, :]
                o = jnp.dot(c, wo_ref[...].astype(BF),
                            preferred_element_type=jnp.float32)
                out_ref[bs, rows, :] = o

        def attn_block(qrows, kbh, vbh, bias):
            s = lax.dot_general(
                qrows, kbh, (((1,), (1,)), ((), ())),
                preferred_element_type=jnp.float32,
            )
            w = jnp.exp((s + bias).astype(BF))
            l = jnp.sum(w, axis=1, keepdims=True, dtype=jnp.float32)
            o = jnp.dot(w, vbh, preferred_element_type=jnp.float32)
            return o, l

        def combine_hi(b):
            p = o_buf[0, b, HALF:, :] + o_buf[1, b, HALF:, :]
            for h in range(HQ):
                hs = slice(h * DH, (h + 1) * DH)
                ctx_buf[b, HALF:, hs] = p[:, hs] / p[:, D + h:D + h + 1]

        @pl.when(is01)
        def _producer():
            partner = 1 - my_pos
            rd_ex = [
                pltpu.make_async_remote_copy(
                    src_ref=o_buf.at[0, b, pl.ds(HALF, HALF), :],
                    dst_ref=o_buf.at[1, b, pl.ds(HALF, HALF), :],
                    send_sem=ex_send.at[b], recv_sem=ex_recv.at[b],
                    device_id=(partner,),
                    device_id_type=pl.DeviceIdType.MESH,
                )
                for b in range(B)
            ]
            rd_lo3 = [
                pltpu.make_async_remote_copy(
                    src_ref=ctx_buf.at[b, pl.ds(0, HALF), :],
                    dst_ref=ctx_buf.at[b, pl.ds(0, HALF), :],
                    send_sem=cx_send.at[b], recv_sem=cx_recv.at[b],
                    device_id=(3,), device_id_type=pl.DeviceIdType.MESH,
                )
                for b in range(B)
            ]
            rd_lo1 = pltpu.make_async_remote_copy(
                src_ref=ctx_buf.at[:, pl.ds(0, HALF), :],
                dst_ref=ctx_buf.at[:, pl.ds(0, HALF), :],
                send_sem=cx_send.at[2], recv_sem=cx_recv.at[0],
                device_id=(1,), device_id_type=pl.DeviceIdType.MESH,
            )

            @pl.when(my_pos == 0)
            def _p0():
                ti = lax.broadcasted_iota(jnp.int32, (HALF, SKV), 0)
                tj = lax.broadcasted_iota(jnp.int32, (HALF, SKV), 1)
                bias = jnp.where(tj >= ti, 0.0, NEG)
                xhi = x_ref[:, HALF:, :].reshape(B * HALF, 512).astype(BF)
                qhi = (jnp.dot(xhi, wq_ref[...].astype(BF),
                               preferred_element_type=jnp.float32)
                       * 0.125).astype(BF)
                for b in range(B):
                    kb = k_ref[b]
                    vb = v_ref[b]
                    for h in range(HQ):
                        hs = slice(h * DH, (h + 1) * DH)
                        o, l = attn_block(
                            qhi[b * HALF:(b + 1) * HALF, hs],
                            kb[:, hs], vb[:, hs], bias)
                        o_buf[0, b, HALF:, hs] = o.astype(BF)
                        o_buf[0, b, HALF:, D + h:D + h + 1] = l.astype(BF)
                    rd_ex[b].start()

                bias_lo = jnp.where(tj - ti <= 128, 0.0, NEG)
                xlo = x_ref[:, 0:HALF, :].reshape(B * HALF, 512).astype(BF)
                qlo = (jnp.dot(xlo, wq_ref[...].astype(BF),
                               preferred_element_type=jnp.float32)
                       * 0.125).astype(BF)
                for b in range(B):
                    kb = k_ref[b]
                    vb = v_ref[b]
                    for h in range(HQ):
                        hs = slice(h * DH, (h + 1) * DH)
                        o, l = attn_block(
                            qlo[b * HALF:(b + 1) * HALF, hs],
                            kb[:, hs], vb[:, hs], bias_lo)
                        ctx_buf[b, 0:HALF, hs] = (o / l).astype(BF)
                    rd_lo3[b].start()
                rd_lo1.start()

                rd_ex[0].wait()
                combine_hi(0)
                rd_ex[1].wait()
                combine_hi(1)
                rd_hi3 = pltpu.make_async_remote_copy(
                    src_ref=ctx_buf.at[:, pl.ds(HALF, HALF), :],
                    dst_ref=ctx_buf.at[:, pl.ds(HALF, HALF), :],
                    send_sem=cx_send.at[3], recv_sem=cx_recv.at[2],
                    device_id=(3,), device_id_type=pl.DeviceIdType.MESH,
                )
                rd_hi3.start()
                proj(slice(0, SQ))
                rd_lo3[0].wait_send()
                rd_lo3[1].wait_send()
                rd_lo1.wait_send()
                rd_hi3.wait_send()

            @pl.when(my_pos == 1)
            def _p1():
                ti = lax.broadcasted_iota(jnp.int32, (HALF, HALF), 0)
                tj = lax.broadcasted_iota(jnp.int32, (HALF, HALF), 1)
                bias = jnp.where(tj <= ti, 0.0, NEG)
                xhi = x_ref[:, HALF:, :].reshape(B * HALF, 512).astype(BF)
                qhi = (jnp.dot(xhi, wq_ref[...].astype(BF),
                               preferred_element_type=jnp.float32)
                       * 0.125).astype(BF)
                for b in range(B):
                    kb = k_ref[b, 0:HALF, :]
                    vb = v_ref[b, 0:HALF, :]
                    for h in range(HQ):
                        hs = slice(h * DH, (h + 1) * DH)
                        o, l = attn_block(
                            qhi[b * HALF:(b + 1) * HALF, hs],
                            kb[:, hs], vb[:, hs], bias)
                        o_buf[0, b, HALF:, hs] = o.astype(BF)
                        o_buf[0, b, HALF:, D + h:D + h + 1] = l.astype(BF)
                    rd_ex[b].start()

                rd_hi2 = [
                    pltpu.make_async_remote_copy(
                        src_ref=ctx_buf.at[b, pl.ds(HALF, HALF), :],
                        dst_ref=ctx_buf.at[b, pl.ds(HALF, HALF), :],
                        send_sem=cx_send.at[b],
                        recv_sem=cx_recv.at[2 + b],
                        device_id=(2,), device_id_type=pl.DeviceIdType.MESH,
                    )
                    for b in range(B)
                ]
                for b in range(B):
                    rd_ex[b].wait()
                    combine_hi(b)
                    rd_hi2[b].start()
                    proj(slice(HALF, SQ), bs=b)
                rd_lo1.wait_recv()
                proj(slice(0, HALF))
                rd_hi2[0].wait_send()
                rd_hi2[1].wait_send()

        @pl.when(jnp.logical_not(is01))
        def _consumer():
            rd_lo = [
                pltpu.make_async_remote_copy(
                    src_ref=ctx_buf.at[b, pl.ds(0, HALF), :],
                    dst_ref=ctx_buf.at[b, pl.ds(0, HALF), :],
                    send_sem=cx_send.at[b], recv_sem=cx_recv.at[b],
                    device_id=(left,), device_id_type=pl.DeviceIdType.MESH,
                )
                for b in range(B)
            ]

            @pl.when(my_pos == 3)
            def _c3():
                rd_hi = pltpu.make_async_remote_copy(
                    src_ref=ctx_buf.at[:, pl.ds(HALF, HALF), :],
                    dst_ref=ctx_buf.at[:, pl.ds(HALF, HALF), :],
                    send_sem=cx_send.at[3], recv_sem=cx_recv.at[2],
                    device_id=(0,), device_id_type=pl.DeviceIdType.MESH,
                )
                rd_fwd = [
                    pltpu.make_async_remote_copy(
                        src_ref=ctx_buf.at[b, pl.ds(0, HALF), :],
                        dst_ref=ctx_buf.at[b, pl.ds(0, HALF), :],
                        send_sem=cx_send.at[b], recv_sem=cx_recv.at[b],
                        device_id=(2,), device_id_type=pl.DeviceIdType.MESH,
                    )
                    for b in range(B)
                ]
                for b in range(B):
                    rd_lo[b].wait_recv()
                    rd_fwd[b].start()
                    proj(slice(0, HALF), bs=b)
                rd_hi.wait_recv()
                proj(slice(HALF, SQ))
                rd_fwd[0].wait_send()
                rd_fwd[1].wait_send()

            @pl.when(my_pos == 2)
            def _c2():
                rd_hi2 = [
                    pltpu.make_async_remote_copy(
                        src_ref=ctx_buf.at[b, pl.ds(HALF, HALF), :],
                        dst_ref=ctx_buf.at[b, pl.ds(HALF, HALF), :],
                        send_sem=cx_send.at[b],
                        recv_sem=cx_recv.at[2 + b],
                        device_id=(1,), device_id_type=pl.DeviceIdType.MESH,
                    )
                    for b in range(B)
                ]
                rd_hi2[0].wait_recv()
                proj(slice(HALF, SQ), bs=0)
                rd_hi2[1].wait_recv()
                proj(slice(HALF, SQ), bs=1)
                rd_lo[0].wait_recv()
                proj(slice(0, HALF), bs=0)
                rd_lo[1].wait_recv()
                proj(slice(0, HALF), bs=1)

    return pl.pallas_call(
        body,
        out_shape=jax.ShapeDtypeStruct((B, SQ, 512), jnp.float32),
        in_specs=[pl.BlockSpec(memory_space=pltpu.VMEM)] * 5,
        out_specs=pl.BlockSpec(memory_space=pltpu.VMEM),
        scratch_shapes=[
            pltpu.VMEM((2, B, SQ, DL), BF),
            pltpu.VMEM((B, SQ, D), BF),
            pltpu.SemaphoreType.DMA((B,)),
            pltpu.SemaphoreType.DMA((B,)),
            pltpu.SemaphoreType.DMA((4,)),
            pltpu.SemaphoreType.DMA((4,)),
        ],
        compiler_params=pltpu.CompilerParams(collective_id=0),
    )(x, Wq, K2, V2, Wo)


# device time: 12000 ns/iter; 4.3582x vs baseline; 1.1354x over previous
import jax
import jax.numpy as jnp
from jax import lax
from jax.experimental import pallas as pl
from jax.experimental.pallas import tpu as pltpu

N_DEV = 4
B = 2
SQ = 256
SKV = 256
HQ = 4
DH = 64
D = HQ * DH
HALF = 128
NEG = -1e9
DL = D + 8
BF = jnp.bfloat16


def kernel(x, Wq, K_ext, V_ext, Wo):
    K2 = K_ext.reshape(B, SKV, D).astype(BF)
    V2 = V_ext.reshape(B, SKV, D).astype(BF)

    def body(x_ref, wq_ref, k_ref, v_ref, wo_ref, out_ref,
             o_buf, ctx_buf, ex_send, ex_recv, cx_send, cx_recv):
        my_pos = lax.axis_index("i")
        left = lax.rem(my_pos + (N_DEV - 1), N_DEV)
        right = lax.rem(my_pos + 1, N_DEV)
        is01 = my_pos <= 1

        barrier_sem = pltpu.get_barrier_semaphore()
        for nbr in (left, right):
            pl.semaphore_signal(
                barrier_sem, inc=1,
                device_id=(nbr,), device_id_type=pl.DeviceIdType.MESH,
            )
        pl.semaphore_wait(barrier_sem, 2)

        def proj(rows, bs=None):
            n = rows.stop - rows.start
            if bs is None:
                c = ctx_buf[:, rows, :].reshape(B * n, D)
                o = jnp.dot(c, wo_ref[...].astype(BF),
                            preferred_element_type=jnp.float32)
                out_ref[:, rows, :] = o.reshape(B, n, 512)
            else:
                c = ctx_buf[bs, rows, :]
                o = jnp.dot(c, wo_ref[...].astype(BF),
                            preferred_element_type=jnp.float32)
                out_ref[bs, rows, :] = o

        def attn_block(qrows, kbh, vbh, bias):
            s = lax.dot_general(
                qrows, kbh, (((1,), (1,)), ((), ())),
                preferred_element_type=jnp.float32,
            )
            w = jnp.exp((s + bias).astype(BF))
            l = jnp.sum(w, axis=1, keepdims=True, dtype=jnp.float32)
            o = jnp.dot(w, vbh, preferred_element_type=jnp.float32)
            return o, l

        def combine_hi(b):
            p = o_buf[0, b, HALF:, :] + o_buf[1, b, HALF:, :]
            for h in range(HQ):
                hs = slice(h * DH, (h + 1) * DH)
                ctx_buf[b, HALF:, hs] = p[:, hs] / p[:, D + h:D + h + 1]

        @pl.when(is01)
        def _producer():
            partner = 1 - my_pos
            rd_ex = [
                pltpu.make_async_remote_copy(
                    src_ref=o_buf.at[0, b, pl.ds(HALF, HALF), :],
                    dst_ref=o_buf.at[1, b, pl.ds(HALF, HALF), :],
                    send_sem=ex_send.at[b], recv_sem=ex_recv.at[b],
                    device_id=(partner,),
                    device_id_type=pl.DeviceIdType.MESH,
                )
                for b in range(B)
            ]
            rd_lo3 = [
                pltpu.make_async_remote_copy(
                    src_ref=ctx_buf.at[b, pl.ds(0, HALF), :],
                    dst_ref=ctx_buf.at[b, pl.ds(0, HALF), :],
                    send_sem=cx_send.at[b], recv_sem=cx_recv.at[b],
                    device_id=(3,), device_id_type=pl.DeviceIdType.MESH,
                )
                for b in range(B)
            ]
            rd_lo1 = [
                pltpu.make_async_remote_copy(
                    src_ref=ctx_buf.at[b, pl.ds(0, HALF), :],
                    dst_ref=ctx_buf.at[b, pl.ds(0, HALF), :],
                    send_sem=cx_send.at[2 + b], recv_sem=cx_recv.at[b],
                    device_id=(1,), device_id_type=pl.DeviceIdType.MESH,
                )
                for b in range(B)
            ]

            @pl.when(my_pos == 0)
            def _p0():
                ti = lax.broadcasted_iota(jnp.int32, (HALF, SKV), 0)
                tj = lax.broadcasted_iota(jnp.int32, (HALF, SKV), 1)
                bias = jnp.where(tj >= ti, 0.0, NEG)
                xhi = x_ref[:, HALF:, :].reshape(B * HALF, 512).astype(BF)
                qhi = (jnp.dot(xhi, wq_ref[...].astype(BF),
                               preferred_element_type=jnp.float32)
                       * 0.125).astype(BF)
                for b in range(B):
                    kb = k_ref[b]
                    vb = v_ref[b]
                    for h in range(HQ):
                        hs = slice(h * DH, (h + 1) * DH)
                        o, l = attn_block(
                            qhi[b * HALF:(b + 1) * HALF, hs],
                            kb[:, hs], vb[:, hs], bias)
                        o_buf[0, b, HALF:, hs] = o.astype(BF)
                        o_buf[0, b, HALF:, D + h:D + h + 1] = l.astype(BF)
                    rd_ex[b].start()

                bias_lo = jnp.where(tj - ti <= 128, 0.0, NEG)
                xlo = x_ref[:, 0:HALF, :].reshape(B * HALF, 512).astype(BF)
                qlo = (jnp.dot(xlo, wq_ref[...].astype(BF),
                               preferred_element_type=jnp.float32)
                       * 0.125).astype(BF)
                for b in range(B):
                    kb = k_ref[b]
                    vb = v_ref[b]
                    for h in range(HQ):
                        hs = slice(h * DH, (h + 1) * DH)
                        o, l = attn_block(
                            qlo[b * HALF:(b + 1) * HALF, hs],
                            kb[:, hs], vb[:, hs], bias_lo)
                        ctx_buf[b, 0:HALF, hs] = (o / l).astype(BF)
                    rd_lo3[b].start()
                    rd_lo1[b].start()

                rd_hi3 = [
                    pltpu.make_async_remote_copy(
                        src_ref=ctx_buf.at[b, pl.ds(HALF, HALF), :],
                        dst_ref=ctx_buf.at[b, pl.ds(HALF, HALF), :],
                        send_sem=cx_send.at[4 + b], recv_sem=cx_recv.at[2 + b],
                        device_id=(3,), device_id_type=pl.DeviceIdType.MESH,
                    )
                    for b in range(B)
                ]
                for b in range(B):
                    rd_ex[b].wait()
                    combine_hi(b)
                    rd_hi3[b].start()
                proj(slice(0, SQ))
                for b in range(B):
                    rd_lo3[b].wait_send()
                    rd_lo1[b].wait_send()
                    rd_hi3[b].wait_send()

            @pl.when(my_pos == 1)
            def _p1():
                ti = lax.broadcasted_iota(jnp.int32, (HALF, HALF), 0)
                tj = lax.broadcasted_iota(jnp.int32, (HALF, HALF), 1)
                bias = jnp.where(tj <= ti, 0.0, NEG)
                xhi = x_ref[:, HALF:, :].reshape(B * HALF, 512).astype(BF)
                qhi = (jnp.dot(xhi, wq_ref[...].astype(BF),
                               preferred_element_type=jnp.float32)
                       * 0.125).astype(BF)
                for b in range(B):
                    kb = k_ref[b, 0:HALF, :]
                    vb = v_ref[b, 0:HALF, :]
                    for h in range(HQ):
                        hs = slice(h * DH, (h + 1) * DH)
                        o, l = attn_block(
                            qhi[b * HALF:(b + 1) * HALF, hs],
                            kb[:, hs], vb[:, hs], bias)
                        o_buf[0, b, HALF:, hs] = o.astype(BF)
                        o_buf[0, b, HALF:, D + h:D + h + 1] = l.astype(BF)
                    rd_ex[b].start()

                rd_hi2 = [
                    pltpu.make_async_remote_copy(
                        src_ref=ctx_buf.at[b, pl.ds(HALF, HALF), :],
                        dst_ref=ctx_buf.at[b, pl.ds(HALF, HALF), :],
                        send_sem=cx_send.at[b],
                        recv_sem=cx_recv.at[2 + b],
                        device_id=(2,), device_id_type=pl.DeviceIdType.MESH,
                    )
                    for b in range(B)
                ]
                for b in range(B):
                    rd_ex[b].wait()
                    combine_hi(b)
                    rd_hi2[b].start()
                    proj(slice(HALF, SQ), bs=b)
                for b in range(B):
                    rd_lo1[b].wait_recv()
                    proj(slice(0, HALF), bs=b)
                rd_hi2[0].wait_send()
                rd_hi2[1].wait_send()

        @pl.when(jnp.logical_not(is01))
        def _consumer():
            rd_lo = [
                pltpu.make_async_remote_copy(
                    src_ref=ctx_buf.at[b, pl.ds(0, HALF), :],
                    dst_ref=ctx_buf.at[b, pl.ds(0, HALF), :],
                    send_sem=cx_send.at[b], recv_sem=cx_recv.at[b],
                    device_id=(left,), device_id_type=pl.DeviceIdType.MESH,
                )
                for b in range(B)
            ]

            @pl.when(my_pos == 3)
            def _c3():
                rd_hi = [
                    pltpu.make_async_remote_copy(
                        src_ref=ctx_buf.at[b, pl.ds(HALF, HALF), :],
                        dst_ref=ctx_buf.at[b, pl.ds(HALF, HALF), :],
                        send_sem=cx_send.at[4 + b], recv_sem=cx_recv.at[2 + b],
                        device_id=(0,), device_id_type=pl.DeviceIdType.MESH,
                    )
                    for b in range(B)
                ]
                rd_fwd = [
                    pltpu.make_async_remote_copy(
                        src_ref=ctx_buf.at[b, pl.ds(0, HALF), :],
                        dst_ref=ctx_buf.at[b, pl.ds(0, HALF), :],
                        send_sem=cx_send.at[b], recv_sem=cx_recv.at[b],
                        device_id=(2,), device_id_type=pl.DeviceIdType.MESH,
                    )
                    for b in range(B)
                ]
                for b in range(B):
                    rd_lo[b].wait_recv()
                    rd_fwd[b].start()
                    proj(slice(0, HALF), bs=b)
                for b in range(B):
                    rd_hi[b].wait_recv()
                    proj(slice(HALF, SQ), bs=b)
                rd_fwd[0].wait_send()
                rd_fwd[1].wait_send()

            @pl.when(my_pos == 2)
            def _c2():
                rd_hi2 = [
                    pltpu.make_async_remote_copy(
                        src_ref=ctx_buf.at[b, pl.ds(HALF, HALF), :],
                        dst_ref=ctx_buf.at[b, pl.ds(HALF, HALF), :],
                        send_sem=cx_send.at[b],
                        recv_sem=cx_recv.at[2 + b],
                        device_id=(1,), device_id_type=pl.DeviceIdType.MESH,
                    )
                    for b in range(B)
                ]
                rd_hi2[0].wait_recv()
                proj(slice(HALF, SQ), bs=0)
                rd_hi2[1].wait_recv()
                proj(slice(HALF, SQ), bs=1)
                rd_lo[0].wait_recv()
                proj(slice(0, HALF), bs=0)
                rd_lo[1].wait_recv()
                proj(slice(0, HALF), bs=1)

    return pl.pallas_call(
        body,
        out_shape=jax.ShapeDtypeStruct((B, SQ, 512), jnp.float32),
        in_specs=[pl.BlockSpec(memory_space=pltpu.VMEM)] * 5,
        out_specs=pl.BlockSpec(memory_space=pltpu.VMEM),
        scratch_shapes=[
            pltpu.VMEM((2, B, SQ, DL), BF),
            pltpu.VMEM((B, SQ, D), BF),
            pltpu.SemaphoreType.DMA((B,)),
            pltpu.SemaphoreType.DMA((B,)),
            pltpu.SemaphoreType.DMA((6,)),
            pltpu.SemaphoreType.DMA((4,)),
        ],
        compiler_params=pltpu.CompilerParams(collective_id=0),
    )(x, Wq, K2, V2, Wo)
